# baseline (speedup 1.0000x reference)
import jax
import jax.numpy as jnp
from jax import lax
from jax.experimental import pallas as pl
from jax.experimental.pallas import tpu as pltpu
from jax.experimental.pallas import tpu_sc as plsc

D = 1024
H = 2048
E = 16
T = 2048
P = 2 * T
BT = 192
G = 36
GP = G * BT

_INTERPRET = False


def _route_kernel(x_ref, gw_ref, gb_ref, pos_ref, w_ref, be_ref):
    f32 = jnp.float32
    xf = x_ref[...]
    logits = jax.lax.dot_general(
        xf, gw_ref[...], (((1,), (1,)), ((), ())),
        preferred_element_type=f32) + gb_ref[...][None, :]
    s = (logits > 1.0).astype(jnp.int32)
    e_iota = jax.lax.broadcasted_iota(jnp.int32, (T, E), 1)
    f1 = e_iota + (1 - s) * E
    m1 = jnp.min(f1, axis=1)
    idx1 = jnp.where(m1 < E, m1, 0)
    v1 = (m1 < E).astype(f32)
    f2 = f1 + jnp.where(e_iota == idx1[:, None], 16 * E, 0)
    m2 = jnp.min(f2, axis=1)
    idx2 = jnp.where(m2 < E, m2, m2 - E)
    v2 = (m2 < E).astype(f32)
    w1 = 1.0 / (1.0 + jnp.exp(v2 - v1))
    w2 = 1.0 - w1

    oh1 = (idx1[:, None] == e_iota).astype(f32)
    oh2 = (idx2[:, None] == e_iota).astype(f32)
    oh = jnp.concatenate([oh1, oh2], axis=0)

    RB = 256
    nb = P // RB
    ltb = (jax.lax.broadcasted_iota(jnp.int32, (RB, RB), 1)
           < jax.lax.broadcasted_iota(jnp.int32, (RB, RB), 0)).astype(f32)
    parts = []
    sums = []
    for b in range(nb):
        ohb = oh[b * RB:(b + 1) * RB]
        parts.append(jnp.dot(ltb, ohb, preferred_element_type=f32))
        sums.append(jnp.sum(ohb, axis=0)[None, :])
    excl_in = jnp.concatenate(parts, axis=0)
    bsums = jnp.concatenate(sums, axis=0)
    ltn = (jax.lax.broadcasted_iota(jnp.int32, (nb, nb), 1)
           < jax.lax.broadcasted_iota(jnp.int32, (nb, nb), 0)).astype(f32)
    bpre = jnp.dot(ltn, bsums, preferred_element_type=f32)
    bases = [jnp.broadcast_to(bpre[b][None, :], (RB, E)) for b in range(nb)]
    excl = excl_in + jnp.concatenate(bases, axis=0)
    rank = jnp.sum(oh * excl, axis=1)

    counts = jnp.sum(oh, axis=0)
    nblk = jnp.floor((counts + (BT - 1)) * (1.0 / BT))
    lte = (jax.lax.broadcasted_iota(jnp.int32, (E, E), 1)
           < jax.lax.broadcasted_iota(jnp.int32, (E, E), 0)).astype(f32)
    blk_start = jnp.dot(lte, nblk[:, None],
                        preferred_element_type=f32)[:, 0]
    pad_off = blk_start * BT
    pos = rank + jnp.sum(oh * pad_off[None, :], axis=1)

    blk_end = blk_start + nblk
    b_iota = jax.lax.broadcasted_iota(jnp.int32, (64, E), 0).astype(f32)
    be = jnp.sum((blk_end[None, :] <= b_iota).astype(f32), axis=1)
    be = jnp.minimum(be, float(E - 1))

    pos_ref[...] = pos.astype(jnp.int32)
    wcat = jnp.concatenate([w1, w2], axis=0)
    w_ref[...] = jnp.broadcast_to(wcat[:, None], (P, 16))
    be_ref[...] = be.astype(jnp.int32)


def _route(xf, gate_W, gate_b):
    return pl.pallas_call(
        _route_kernel,
        out_shape=(
            jax.ShapeDtypeStruct((P,), jnp.int32),
            jax.ShapeDtypeStruct((P, 16), jnp.float32),
            jax.ShapeDtypeStruct((64,), jnp.int32),
        ),
        interpret=_INTERPRET,
    )(xf, gate_W, gate_b)


def _ffn_kernel(be_ref, xs_ref, wg_ref, bg_ref, wu_ref, bu_ref,
                wd_ref, bd_ref, ys_ref):
    f32 = jnp.float32
    xb = xs_ref[...]
    h = jax.lax.dot_general(
        xb, wg_ref[0], (((1,), (1,)), ((), ())),
        preferred_element_type=f32) + bg_ref[0]
    sp = (h > 1.0).astype(f32)
    up = jax.lax.dot_general(
        xb, wu_ref[0], (((1,), (1,)), ((), ())),
        preferred_element_type=f32) + bu_ref[0]
    prod = sp * up
    ys_ref[...] = jax.lax.dot_general(
        prod, wd_ref[0], (((1,), (1,)), ((), ())),
        preferred_element_type=f32) + bd_ref[0]


def _ffn(be, xs, Wg, bg, Wu, bu, Wd, bd):
    grid_spec = pltpu.PrefetchScalarGridSpec(
        num_scalar_prefetch=1,
        grid=(G,),
        in_specs=[
            pl.BlockSpec((BT, D), lambda b, be: (b, 0)),
            pl.BlockSpec((1, H, D), lambda b, be: (be[b], 0, 0)),
            pl.BlockSpec((1, 1, H), lambda b, be: (be[b], 0, 0)),
            pl.BlockSpec((1, H, D), lambda b, be: (be[b], 0, 0)),
            pl.BlockSpec((1, 1, H), lambda b, be: (be[b], 0, 0)),
            pl.BlockSpec((1, D, H), lambda b, be: (be[b], 0, 0)),
            pl.BlockSpec((1, 1, D), lambda b, be: (be[b], 0, 0)),
        ],
        out_specs=pl.BlockSpec((BT, D), lambda b, be: (b, 0)),
    )
    return pl.pallas_call(
        _ffn_kernel,
        grid_spec=grid_spec,
        out_shape=jax.ShapeDtypeStruct((GP, D), jnp.float32),
        compiler_params=pltpu.CompilerParams(
            dimension_semantics=("arbitrary",),
            vmem_limit_bytes=64 * 1024 * 1024,
        ),
        interpret=_INTERPRET,
    )(be, xs, Wg, bg.reshape(E, 1, H),
      Wu, bu.reshape(E, 1, H),
      Wd, bd.reshape(E, 1, D))


def _sc_disperse(pos2d, xf):
    mesh = plsc.VectorSubcoreMesh(core_axis_name="c", subcore_axis_name="s")

    def body(pos_hbm, x_hbm, xs_hbm, posv, rows, sem):
        cid = lax.axis_index("c")
        sid = lax.axis_index("s")
        wid = sid * 2 + cid
        tok0 = (wid * 128) % T
        pltpu.sync_copy(pos_hbm.at[pl.ds(wid * 2, 2)], posv)
        for c in range(2):
            pltpu.sync_copy(x_hbm.at[pl.ds(tok0 + c * 64, 64)], rows)
            pltpu.async_copy(rows, xs_hbm.at[posv.at[c]], sem).wait()

    return pl.kernel(
        body,
        mesh=mesh,
        out_type=jax.ShapeDtypeStruct((GP, D), jnp.float32),
        scratch_types=[
            pltpu.VMEM((2, 64), jnp.int32),
            pltpu.VMEM((64, D), jnp.float32),
            pltpu.SemaphoreType.DMA,
        ],
    )(pos2d, xf)


def _sc_combine(pos, wexp, ys):
    mesh = plsc.VectorSubcoreMesh(core_axis_name="c", subcore_axis_name="s")

    def body(pos_hbm, w_hbm, ys_hbm, out_hbm, i1, i2, w1s, w2s, r1, r2, o,
             sem):
        cid = lax.axis_index("c")
        sid = lax.axis_index("s")
        wid = sid * 2 + cid
        base = wid * 64
        pltpu.sync_copy(pos_hbm.at[pl.ds(base, 64)], i1)
        pltpu.sync_copy(pos_hbm.at[pl.ds(T + base, 64)], i2)
        pltpu.sync_copy(w_hbm.at[pl.ds(base, 64)], w1s)
        pltpu.sync_copy(w_hbm.at[pl.ds(T + base, 64)], w2s)
        lanes = pl.ds(0, 16)
        for c in range(2):
            pltpu.async_copy(ys_hbm.at[i1.at[pl.ds(c * 32, 32)]], r1,
                             sem).wait()
            pltpu.async_copy(ys_hbm.at[i2.at[pl.ds(c * 32, 32)]], r2,
                             sem).wait()

            def tok(t, carry):
                a1 = w1s[c * 32 + t, lanes]
                a2 = w2s[c * 32 + t, lanes]
                for v in range(D // 16):
                    sl = pl.ds(v * 16, 16)
                    o[t, sl] = a1 * r1[t, sl] + a2 * r2[t, sl]
                return carry

            lax.fori_loop(0, 32, tok, 0)
            pltpu.sync_copy(o, out_hbm.at[pl.ds(base + c * 32, 32)])

    return pl.kernel(
        body,
        mesh=mesh,
        out_type=jax.ShapeDtypeStruct((T, D), jnp.float32),
        scratch_types=[
            pltpu.VMEM((64,), jnp.int32),
            pltpu.VMEM((64,), jnp.int32),
            pltpu.VMEM((64, 16), jnp.float32),
            pltpu.VMEM((64, 16), jnp.float32),
            pltpu.VMEM((32, D), jnp.float32),
            pltpu.VMEM((32, D), jnp.float32),
            pltpu.VMEM((32, D), jnp.float32),
            pltpu.SemaphoreType.DMA,
        ],
    )(pos, wexp, ys)


def kernel(x, gate_W, gate_b, Wg, bg, Wu, bu, Wd, bd):
    B, S, _ = x.shape
    xf = x.reshape(T, D)
    pos, wexp, be = _route(xf, gate_W, gate_b)

    xs = _sc_disperse(pos.reshape(64, 64), xf)
    ys = _ffn(be, xs, Wg, bg, Wu, bu, Wd, bd)
    out = _sc_combine(pos, wexp, ys)
    return out.reshape(B, S, D)


# speedup vs baseline: 1.1063x; 1.1063x over previous
import jax
import jax.numpy as jnp
from jax import lax
from jax.experimental import pallas as pl
from jax.experimental.pallas import tpu as pltpu
from jax.experimental.pallas import tpu_sc as plsc

D = 1024
H = 2048
E = 16
T = 2048
P = 2 * T
BT = 384
G = 26
GP = G * BT

_INTERPRET = False


def _route_kernel(x_ref, gw_ref, gb_ref, pos_ref, w_ref, be_ref):
    f32 = jnp.float32
    xf = x_ref[...]
    logits = jax.lax.dot_general(
        xf, gw_ref[...], (((1,), (1,)), ((), ())),
        preferred_element_type=f32) + gb_ref[...][None, :]
    s = (logits > 1.0).astype(jnp.int32)
    e_iota = jax.lax.broadcasted_iota(jnp.int32, (T, E), 1)
    f1 = e_iota + (1 - s) * E
    m1 = jnp.min(f1, axis=1)
    idx1 = jnp.where(m1 < E, m1, 0)
    v1 = (m1 < E).astype(f32)
    f2 = f1 + jnp.where(e_iota == idx1[:, None], 16 * E, 0)
    m2 = jnp.min(f2, axis=1)
    idx2 = jnp.where(m2 < E, m2, m2 - E)
    v2 = (m2 < E).astype(f32)
    w1 = 1.0 / (1.0 + jnp.exp(v2 - v1))
    w2 = 1.0 - w1

    oh1 = (idx1[:, None] == e_iota).astype(f32)
    oh2 = (idx2[:, None] == e_iota).astype(f32)
    oh = jnp.concatenate([oh1, oh2], axis=0)

    RB = 256
    nb = P // RB
    ltb = (jax.lax.broadcasted_iota(jnp.int32, (RB, RB), 1)
           < jax.lax.broadcasted_iota(jnp.int32, (RB, RB), 0)).astype(f32)
    parts = []
    sums = []
    for b in range(nb):
        ohb = oh[b * RB:(b + 1) * RB]
        parts.append(jnp.dot(ltb, ohb, preferred_element_type=f32))
        sums.append(jnp.sum(ohb, axis=0)[None, :])
    excl_in = jnp.concatenate(parts, axis=0)
    bsums = jnp.concatenate(sums, axis=0)
    ltn = (jax.lax.broadcasted_iota(jnp.int32, (nb, nb), 1)
           < jax.lax.broadcasted_iota(jnp.int32, (nb, nb), 0)).astype(f32)
    bpre = jnp.dot(ltn, bsums, preferred_element_type=f32)
    bases = [jnp.broadcast_to(bpre[b][None, :], (RB, E)) for b in range(nb)]
    excl = excl_in + jnp.concatenate(bases, axis=0)
    rank = jnp.sum(oh * excl, axis=1)

    counts = jnp.sum(oh, axis=0)
    nblk = jnp.floor((counts + (BT - 1)) * (1.0 / BT))
    lte = (jax.lax.broadcasted_iota(jnp.int32, (E, E), 1)
           < jax.lax.broadcasted_iota(jnp.int32, (E, E), 0)).astype(f32)
    blk_start = jnp.dot(lte, nblk[:, None],
                        preferred_element_type=f32)[:, 0]
    pad_off = blk_start * BT
    pos = rank + jnp.sum(oh * pad_off[None, :], axis=1)

    blk_end = blk_start + nblk
    b_iota = jax.lax.broadcasted_iota(jnp.int32, (64, E), 0).astype(f32)
    be = jnp.sum((blk_end[None, :] <= b_iota).astype(f32), axis=1)
    be = jnp.minimum(be, float(E - 1))

    pos_ref[...] = pos.astype(jnp.int32)
    wcat = jnp.concatenate([w1, w2], axis=0)
    w_ref[...] = jnp.broadcast_to(wcat[:, None], (P, 16))
    be_ref[...] = be.astype(jnp.int32)


def _route(xf, gate_W, gate_b):
    return pl.pallas_call(
        _route_kernel,
        out_shape=(
            jax.ShapeDtypeStruct((P,), jnp.int32),
            jax.ShapeDtypeStruct((P, 16), jnp.float32),
            jax.ShapeDtypeStruct((64,), jnp.int32),
        ),
        interpret=_INTERPRET,
    )(xf, gate_W, gate_b)


def _ffn_kernel(be_ref, xs_ref, wg_ref, bg_ref, wu_ref, bu_ref,
                wd_ref, bd_ref, ys_ref):
    f32 = jnp.float32
    xb = xs_ref[...]
    h = jax.lax.dot_general(
        xb, wg_ref[0], (((1,), (1,)), ((), ())),
        preferred_element_type=f32) + bg_ref[0]
    sp = (h > 1.0).astype(f32)
    up = jax.lax.dot_general(
        xb, wu_ref[0], (((1,), (1,)), ((), ())),
        preferred_element_type=f32) + bu_ref[0]
    prod = sp * up
    ys_ref[...] = jax.lax.dot_general(
        prod, wd_ref[0], (((1,), (1,)), ((), ())),
        preferred_element_type=f32) + bd_ref[0]


def _ffn(be, xs, Wg, bg, Wu, bu, Wd, bd):
    grid_spec = pltpu.PrefetchScalarGridSpec(
        num_scalar_prefetch=1,
        grid=(G,),
        in_specs=[
            pl.BlockSpec((BT, D), lambda b, be: (b, 0)),
            pl.BlockSpec((1, H, D), lambda b, be: (be[b], 0, 0)),
            pl.BlockSpec((1, 1, H), lambda b, be: (be[b], 0, 0)),
            pl.BlockSpec((1, H, D), lambda b, be: (be[b], 0, 0)),
            pl.BlockSpec((1, 1, H), lambda b, be: (be[b], 0, 0)),
            pl.BlockSpec((1, D, H), lambda b, be: (be[b], 0, 0)),
            pl.BlockSpec((1, 1, D), lambda b, be: (be[b], 0, 0)),
        ],
        out_specs=pl.BlockSpec((BT, D), lambda b, be: (b, 0)),
    )
    return pl.pallas_call(
        _ffn_kernel,
        grid_spec=grid_spec,
        out_shape=jax.ShapeDtypeStruct((GP, D), jnp.float32),
        compiler_params=pltpu.CompilerParams(
            dimension_semantics=("arbitrary",),
            vmem_limit_bytes=64 * 1024 * 1024,
        ),
        interpret=_INTERPRET,
    )(be, xs, Wg, bg.reshape(E, 1, H),
      Wu, bu.reshape(E, 1, H),
      Wd, bd.reshape(E, 1, D))


def _sc_disperse(pos2d, xf):
    mesh = plsc.VectorSubcoreMesh(core_axis_name="c", subcore_axis_name="s")

    def body(pos_hbm, x_hbm, xs_hbm, posv, rows, sem):
        cid = lax.axis_index("c")
        sid = lax.axis_index("s")
        wid = sid * 2 + cid
        tok0 = (wid * 128) % T
        pltpu.sync_copy(pos_hbm.at[pl.ds(wid * 2, 2)], posv)
        for c in range(2):
            pltpu.sync_copy(x_hbm.at[pl.ds(tok0 + c * 64, 64)], rows)
            pltpu.async_copy(rows, xs_hbm.at[posv.at[c]], sem).wait()

    return pl.kernel(
        body,
        mesh=mesh,
        out_type=jax.ShapeDtypeStruct((GP, D), jnp.float32),
        scratch_types=[
            pltpu.VMEM((2, 64), jnp.int32),
            pltpu.VMEM((64, D), jnp.float32),
            pltpu.SemaphoreType.DMA,
        ],
    )(pos2d, xf)


def _sc_combine(pos, wexp, ys):
    mesh = plsc.VectorSubcoreMesh(core_axis_name="c", subcore_axis_name="s")

    def body(pos_hbm, w_hbm, ys_hbm, out_hbm, i1, i2, w1s, w2s, r1, r2, o,
             sem):
        cid = lax.axis_index("c")
        sid = lax.axis_index("s")
        wid = sid * 2 + cid
        base = wid * 64
        pltpu.sync_copy(pos_hbm.at[pl.ds(base, 64)], i1)
        pltpu.sync_copy(pos_hbm.at[pl.ds(T + base, 64)], i2)
        pltpu.sync_copy(w_hbm.at[pl.ds(base, 64)], w1s)
        pltpu.sync_copy(w_hbm.at[pl.ds(T + base, 64)], w2s)
        lanes = pl.ds(0, 16)
        for c in range(2):
            pltpu.async_copy(ys_hbm.at[i1.at[pl.ds(c * 32, 32)]], r1,
                             sem).wait()
            pltpu.async_copy(ys_hbm.at[i2.at[pl.ds(c * 32, 32)]], r2,
                             sem).wait()

            def tok(t, carry):
                a1 = w1s[c * 32 + t, lanes]
                a2 = w2s[c * 32 + t, lanes]
                for v in range(D // 16):
                    sl = pl.ds(v * 16, 16)
                    o[t, sl] = a1 * r1[t, sl] + a2 * r2[t, sl]
                return carry

            lax.fori_loop(0, 32, tok, 0)
            pltpu.sync_copy(o, out_hbm.at[pl.ds(base + c * 32, 32)])

    return pl.kernel(
        body,
        mesh=mesh,
        out_type=jax.ShapeDtypeStruct((T, D), jnp.float32),
        scratch_types=[
            pltpu.VMEM((64,), jnp.int32),
            pltpu.VMEM((64,), jnp.int32),
            pltpu.VMEM((64, 16), jnp.float32),
            pltpu.VMEM((64, 16), jnp.float32),
            pltpu.VMEM((32, D), jnp.float32),
            pltpu.VMEM((32, D), jnp.float32),
            pltpu.VMEM((32, D), jnp.float32),
            pltpu.SemaphoreType.DMA,
        ],
    )(pos, wexp, ys)


def kernel(x, gate_W, gate_b, Wg, bg, Wu, bu, Wd, bd):
    B, S, _ = x.shape
    xf = x.reshape(T, D)
    pos, wexp, be = _route(xf, gate_W, gate_b)

    xs = _sc_disperse(pos.reshape(64, 64), xf)
    ys = _ffn(be, xs, Wg, bg, Wu, bu, Wd, bd)
    out = _sc_combine(pos, wexp, ys)
    return out.reshape(B, S, D)


# speedup vs baseline: 1.1633x; 1.0515x over previous
import jax
import jax.numpy as jnp
from jax import lax
from jax.experimental import pallas as pl
from jax.experimental.pallas import tpu as pltpu
from jax.experimental.pallas import tpu_sc as plsc

D = 1024
H = 2048
E = 16
T = 2048
P = 2 * T
BT = 384
G = 26
GP = G * BT

_INTERPRET = False


def _route_kernel(x_ref, gw_ref, gb_ref, pos_ref, w_ref, be_ref):
    f32 = jnp.float32
    xf = x_ref[...]
    logits = jax.lax.dot_general(
        xf, gw_ref[...], (((1,), (1,)), ((), ())),
        preferred_element_type=f32) + gb_ref[...][None, :]
    s = (logits > 1.0).astype(jnp.int32)
    e_iota = jax.lax.broadcasted_iota(jnp.int32, (T, E), 1)
    f1 = e_iota + (1 - s) * E
    m1 = jnp.min(f1, axis=1)
    idx1 = jnp.where(m1 < E, m1, 0)
    v1 = (m1 < E).astype(f32)
    f2 = f1 + jnp.where(e_iota == idx1[:, None], 16 * E, 0)
    m2 = jnp.min(f2, axis=1)
    idx2 = jnp.where(m2 < E, m2, m2 - E)
    v2 = (m2 < E).astype(f32)
    w1 = 1.0 / (1.0 + jnp.exp(v2 - v1))
    w2 = 1.0 - w1

    oh1 = (idx1[:, None] == e_iota).astype(f32)
    oh2 = (idx2[:, None] == e_iota).astype(f32)
    oh = jnp.concatenate([oh1, oh2], axis=0)

    RB = 256
    nb = P // RB
    ltb = (jax.lax.broadcasted_iota(jnp.int32, (RB, RB), 1)
           < jax.lax.broadcasted_iota(jnp.int32, (RB, RB), 0)).astype(f32)
    parts = []
    sums = []
    for b in range(nb):
        ohb = oh[b * RB:(b + 1) * RB]
        parts.append(jnp.dot(ltb, ohb, preferred_element_type=f32))
        sums.append(jnp.sum(ohb, axis=0)[None, :])
    excl_in = jnp.concatenate(parts, axis=0)
    bsums = jnp.concatenate(sums, axis=0)
    ltn = (jax.lax.broadcasted_iota(jnp.int32, (nb, nb), 1)
           < jax.lax.broadcasted_iota(jnp.int32, (nb, nb), 0)).astype(f32)
    bpre = jnp.dot(ltn, bsums, preferred_element_type=f32)
    bases = [jnp.broadcast_to(bpre[b][None, :], (RB, E)) for b in range(nb)]
    excl = excl_in + jnp.concatenate(bases, axis=0)
    rank = jnp.sum(oh * excl, axis=1)

    counts = jnp.sum(oh, axis=0)
    nblk = jnp.floor((counts + (BT - 1)) * (1.0 / BT))
    lte = (jax.lax.broadcasted_iota(jnp.int32, (E, E), 1)
           < jax.lax.broadcasted_iota(jnp.int32, (E, E), 0)).astype(f32)
    blk_start = jnp.dot(lte, nblk[:, None],
                        preferred_element_type=f32)[:, 0]
    pad_off = blk_start * BT
    pos = rank + jnp.sum(oh * pad_off[None, :], axis=1)

    blk_end = blk_start + nblk
    b_iota = jax.lax.broadcasted_iota(jnp.int32, (64, E), 0).astype(f32)
    be = jnp.sum((blk_end[None, :] <= b_iota).astype(f32), axis=1)
    be = jnp.minimum(be, float(E - 1))
    used = jnp.sum(nblk)
    slot = jax.lax.broadcasted_iota(jnp.int32, (64,), 0)
    be = jnp.where(slot == 63, used, be)

    pos_ref[...] = pos.astype(jnp.int32)
    wcat = jnp.concatenate([w1, w2], axis=0)
    w_ref[...] = jnp.broadcast_to(wcat[:, None], (P, 16))
    be_ref[...] = be.astype(jnp.int32)


def _route(xf, gate_W, gate_b):
    return pl.pallas_call(
        _route_kernel,
        out_shape=(
            jax.ShapeDtypeStruct((P,), jnp.int32),
            jax.ShapeDtypeStruct((P, 16), jnp.float32),
            jax.ShapeDtypeStruct((64,), jnp.int32),
        ),
        interpret=_INTERPRET,
    )(xf, gate_W, gate_b)


def _cb(b, be):
    return jnp.minimum(b, be[63] - 1)


def _ffn_kernel(be_ref, xs_ref, wg_ref, bg_ref, wu_ref, bu_ref,
                wd_ref, bd_ref, ys_ref):
    f32 = jnp.float32
    xb = xs_ref[...]
    h = jax.lax.dot_general(
        xb, wg_ref[0], (((1,), (1,)), ((), ())),
        preferred_element_type=f32) + bg_ref[0]
    sp = (h > 1.0).astype(f32)
    up = jax.lax.dot_general(
        xb, wu_ref[0], (((1,), (1,)), ((), ())),
        preferred_element_type=f32) + bu_ref[0]
    prod = sp * up
    ys_ref[...] = jax.lax.dot_general(
        prod, wd_ref[0], (((1,), (1,)), ((), ())),
        preferred_element_type=f32) + bd_ref[0]


def _ffn_kernel_skip(be_ref, xs_ref, wg_ref, bg_ref, wu_ref, bu_ref,
                     wd_ref, bd_ref, ys_ref):
    @pl.when(pl.program_id(0) < be_ref[63])
    def _():
        _ffn_kernel(be_ref, xs_ref, wg_ref, bg_ref, wu_ref, bu_ref,
                    wd_ref, bd_ref, ys_ref)


def _ffn(be, xs, Wg, bg, Wu, bu, Wd, bd):
    grid_spec = pltpu.PrefetchScalarGridSpec(
        num_scalar_prefetch=1,
        grid=(G,),
        in_specs=[
            pl.BlockSpec((BT, D), lambda b, be: (_cb(b, be), 0)),
            pl.BlockSpec((1, H, D), lambda b, be: (be[_cb(b, be)], 0, 0)),
            pl.BlockSpec((1, 1, H), lambda b, be: (be[_cb(b, be)], 0, 0)),
            pl.BlockSpec((1, H, D), lambda b, be: (be[_cb(b, be)], 0, 0)),
            pl.BlockSpec((1, 1, H), lambda b, be: (be[_cb(b, be)], 0, 0)),
            pl.BlockSpec((1, D, H), lambda b, be: (be[_cb(b, be)], 0, 0)),
            pl.BlockSpec((1, 1, D), lambda b, be: (be[_cb(b, be)], 0, 0)),
        ],
        out_specs=pl.BlockSpec((BT, D), lambda b, be: (_cb(b, be), 0)),
    )
    return pl.pallas_call(
        _ffn_kernel_skip,
        grid_spec=grid_spec,
        out_shape=jax.ShapeDtypeStruct((GP, D), jnp.float32),
        compiler_params=pltpu.CompilerParams(
            dimension_semantics=("arbitrary",),
            vmem_limit_bytes=64 * 1024 * 1024,
        ),
        interpret=_INTERPRET,
    )(be, xs, Wg, bg.reshape(E, 1, H),
      Wu, bu.reshape(E, 1, H),
      Wd, bd.reshape(E, 1, D))


def _sc_disperse(pos2d, xf):
    mesh = plsc.VectorSubcoreMesh(core_axis_name="c", subcore_axis_name="s")

    def body(pos_hbm, x_hbm, xs_hbm, posv, rows, sem):
        cid = lax.axis_index("c")
        sid = lax.axis_index("s")
        wid = sid * 2 + cid
        tok0 = (wid * 128) % T
        pltpu.sync_copy(pos_hbm.at[pl.ds(wid * 2, 2)], posv)
        for c in range(2):
            pltpu.sync_copy(x_hbm.at[pl.ds(tok0 + c * 64, 64)], rows)
            pltpu.async_copy(rows, xs_hbm.at[posv.at[c]], sem).wait()

    return pl.kernel(
        body,
        mesh=mesh,
        out_type=jax.ShapeDtypeStruct((GP, D), jnp.float32),
        scratch_types=[
            pltpu.VMEM((2, 64), jnp.int32),
            pltpu.VMEM((64, D), jnp.float32),
            pltpu.SemaphoreType.DMA,
        ],
    )(pos2d, xf)


def _sc_combine(pos, wexp, ys):
    mesh = plsc.VectorSubcoreMesh(core_axis_name="c", subcore_axis_name="s")

    def body(pos_hbm, w_hbm, ys_hbm, out_hbm, i1, i2, w1s, w2s, r1, r2, o,
             sem):
        cid = lax.axis_index("c")
        sid = lax.axis_index("s")
        wid = sid * 2 + cid
        base = wid * 64
        pltpu.sync_copy(pos_hbm.at[pl.ds(base, 64)], i1)
        pltpu.sync_copy(pos_hbm.at[pl.ds(T + base, 64)], i2)
        pltpu.sync_copy(w_hbm.at[pl.ds(base, 64)], w1s)
        pltpu.sync_copy(w_hbm.at[pl.ds(T + base, 64)], w2s)
        lanes = pl.ds(0, 16)
        for c in range(2):
            pltpu.async_copy(ys_hbm.at[i1.at[pl.ds(c * 32, 32)]], r1,
                             sem).wait()
            pltpu.async_copy(ys_hbm.at[i2.at[pl.ds(c * 32, 32)]], r2,
                             sem).wait()

            def tok(t, carry):
                a1 = w1s[c * 32 + t, lanes]
                a2 = w2s[c * 32 + t, lanes]
                for v in range(D // 16):
                    sl = pl.ds(v * 16, 16)
                    o[t, sl] = a1 * r1[t, sl] + a2 * r2[t, sl]
                return carry

            lax.fori_loop(0, 32, tok, 0)
            pltpu.sync_copy(o, out_hbm.at[pl.ds(base + c * 32, 32)])

    return pl.kernel(
        body,
        mesh=mesh,
        out_type=jax.ShapeDtypeStruct((T, D), jnp.float32),
        scratch_types=[
            pltpu.VMEM((64,), jnp.int32),
            pltpu.VMEM((64,), jnp.int32),
            pltpu.VMEM((64, 16), jnp.float32),
            pltpu.VMEM((64, 16), jnp.float32),
            pltpu.VMEM((32, D), jnp.float32),
            pltpu.VMEM((32, D), jnp.float32),
            pltpu.VMEM((32, D), jnp.float32),
            pltpu.SemaphoreType.DMA,
        ],
    )(pos, wexp, ys)


def kernel(x, gate_W, gate_b, Wg, bg, Wu, bu, Wd, bd):
    B, S, _ = x.shape
    xf = x.reshape(T, D)
    pos, wexp, be = _route(xf, gate_W, gate_b)

    xs = _sc_disperse(pos.reshape(64, 64), xf)
    ys = _ffn(be, xs, Wg, bg, Wu, bu, Wd, bd)
    out = _sc_combine(pos, wexp, ys)
    return out.reshape(B, S, D)


# speedup vs baseline: 1.1707x; 1.0064x over previous
import jax
import jax.numpy as jnp
from jax import lax
from jax.experimental import pallas as pl
from jax.experimental.pallas import tpu as pltpu
from jax.experimental.pallas import tpu_sc as plsc

D = 1024
H = 2048
E = 16
T = 2048
P = 2 * T
BT = 256
G = 31
GP = G * BT

_INTERPRET = False


def _route_kernel(x_ref, gw_ref, gb_ref, pos_ref, w_ref, be_ref):
    f32 = jnp.float32
    xf = x_ref[...]
    logits = jax.lax.dot_general(
        xf, gw_ref[...], (((1,), (1,)), ((), ())),
        preferred_element_type=f32) + gb_ref[...][None, :]
    s = (logits > 1.0).astype(jnp.int32)
    e_iota = jax.lax.broadcasted_iota(jnp.int32, (T, E), 1)
    f1 = e_iota + (1 - s) * E
    m1 = jnp.min(f1, axis=1)
    idx1 = jnp.where(m1 < E, m1, 0)
    v1 = (m1 < E).astype(f32)
    f2 = f1 + jnp.where(e_iota == idx1[:, None], 16 * E, 0)
    m2 = jnp.min(f2, axis=1)
    idx2 = jnp.where(m2 < E, m2, m2 - E)
    v2 = (m2 < E).astype(f32)
    w1 = 1.0 / (1.0 + jnp.exp(v2 - v1))
    w2 = 1.0 - w1

    oh1 = (idx1[:, None] == e_iota).astype(f32)
    oh2 = (idx2[:, None] == e_iota).astype(f32)
    oh = jnp.concatenate([oh1, oh2], axis=0)

    RB = 256
    nb = P // RB
    ltb = (jax.lax.broadcasted_iota(jnp.int32, (RB, RB), 1)
           < jax.lax.broadcasted_iota(jnp.int32, (RB, RB), 0)).astype(f32)
    parts = []
    sums = []
    for b in range(nb):
        ohb = oh[b * RB:(b + 1) * RB]
        parts.append(jnp.dot(ltb, ohb, preferred_element_type=f32))
        sums.append(jnp.sum(ohb, axis=0)[None, :])
    excl_in = jnp.concatenate(parts, axis=0)
    bsums = jnp.concatenate(sums, axis=0)
    ltn = (jax.lax.broadcasted_iota(jnp.int32, (nb, nb), 1)
           < jax.lax.broadcasted_iota(jnp.int32, (nb, nb), 0)).astype(f32)
    bpre = jnp.dot(ltn, bsums, preferred_element_type=f32)
    bases = [jnp.broadcast_to(bpre[b][None, :], (RB, E)) for b in range(nb)]
    excl = excl_in + jnp.concatenate(bases, axis=0)
    rank = jnp.sum(oh * excl, axis=1)

    counts = jnp.sum(oh, axis=0)
    nblk = jnp.floor((counts + (BT - 1)) * (1.0 / BT))
    lte = (jax.lax.broadcasted_iota(jnp.int32, (E, E), 1)
           < jax.lax.broadcasted_iota(jnp.int32, (E, E), 0)).astype(f32)
    blk_start = jnp.dot(lte, nblk[:, None],
                        preferred_element_type=f32)[:, 0]
    pad_off = blk_start * BT
    pos = rank + jnp.sum(oh * pad_off[None, :], axis=1)

    blk_end = blk_start + nblk
    b_iota = jax.lax.broadcasted_iota(jnp.int32, (64, E), 0).astype(f32)
    be = jnp.sum((blk_end[None, :] <= b_iota).astype(f32), axis=1)
    be = jnp.minimum(be, float(E - 1))
    used = jnp.sum(nblk)
    slot = jax.lax.broadcasted_iota(jnp.int32, (64,), 0)
    be = jnp.where(slot == 63, used, be)

    pos_ref[...] = pos.astype(jnp.int32)
    wcat = jnp.concatenate([w1, w2], axis=0)
    w_ref[...] = jnp.broadcast_to(wcat[:, None], (P, 16))
    be_ref[...] = be.astype(jnp.int32)


def _route(xf, gate_W, gate_b):
    return pl.pallas_call(
        _route_kernel,
        out_shape=(
            jax.ShapeDtypeStruct((P,), jnp.int32),
            jax.ShapeDtypeStruct((P, 16), jnp.float32),
            jax.ShapeDtypeStruct((64,), jnp.int32),
        ),
        interpret=_INTERPRET,
    )(xf, gate_W, gate_b)


def _cb(b, be):
    return jnp.minimum(b, be[63] - 1)


def _ffn_kernel(be_ref, xs_ref, wg_ref, bg_ref, wu_ref, bu_ref,
                wd_ref, bd_ref, ys_ref):
    f32 = jnp.float32
    xb = xs_ref[...]
    h = jax.lax.dot_general(
        xb, wg_ref[0], (((1,), (1,)), ((), ())),
        preferred_element_type=f32) + bg_ref[0]
    sp = (h > 1.0).astype(f32)
    up = jax.lax.dot_general(
        xb, wu_ref[0], (((1,), (1,)), ((), ())),
        preferred_element_type=f32) + bu_ref[0]
    prod = sp * up
    ys_ref[...] = jax.lax.dot_general(
        prod, wd_ref[0], (((1,), (1,)), ((), ())),
        preferred_element_type=f32) + bd_ref[0]


def _ffn_kernel_skip(be_ref, xs_ref, wg_ref, bg_ref, wu_ref, bu_ref,
                     wd_ref, bd_ref, ys_ref):
    @pl.when(pl.program_id(0) < be_ref[63])
    def _():
        _ffn_kernel(be_ref, xs_ref, wg_ref, bg_ref, wu_ref, bu_ref,
                    wd_ref, bd_ref, ys_ref)


def _ffn(be, xs, Wg, bg, Wu, bu, Wd, bd):
    grid_spec = pltpu.PrefetchScalarGridSpec(
        num_scalar_prefetch=1,
        grid=(G,),
        in_specs=[
            pl.BlockSpec((BT, D), lambda b, be: (_cb(b, be), 0)),
            pl.BlockSpec((1, H, D), lambda b, be: (be[_cb(b, be)], 0, 0)),
            pl.BlockSpec((1, 1, H), lambda b, be: (be[_cb(b, be)], 0, 0)),
            pl.BlockSpec((1, H, D), lambda b, be: (be[_cb(b, be)], 0, 0)),
            pl.BlockSpec((1, 1, H), lambda b, be: (be[_cb(b, be)], 0, 0)),
            pl.BlockSpec((1, D, H), lambda b, be: (be[_cb(b, be)], 0, 0)),
            pl.BlockSpec((1, 1, D), lambda b, be: (be[_cb(b, be)], 0, 0)),
        ],
        out_specs=pl.BlockSpec((BT, D), lambda b, be: (_cb(b, be), 0)),
    )
    return pl.pallas_call(
        _ffn_kernel_skip,
        grid_spec=grid_spec,
        out_shape=jax.ShapeDtypeStruct((GP, D), jnp.float32),
        compiler_params=pltpu.CompilerParams(
            dimension_semantics=("arbitrary",),
            vmem_limit_bytes=64 * 1024 * 1024,
        ),
        interpret=_INTERPRET,
    )(be, xs, Wg, bg.reshape(E, 1, H),
      Wu, bu.reshape(E, 1, H),
      Wd, bd.reshape(E, 1, D))


def _sc_disperse(pos2d, xf):
    mesh = plsc.VectorSubcoreMesh(core_axis_name="c", subcore_axis_name="s")

    def body(pos_hbm, x_hbm, xs_hbm, posv, rows, sem):
        cid = lax.axis_index("c")
        sid = lax.axis_index("s")
        wid = sid * 2 + cid
        tok0 = (wid * 128) % T
        pltpu.sync_copy(pos_hbm.at[pl.ds(wid * 2, 2)], posv)
        for c in range(2):
            pltpu.sync_copy(x_hbm.at[pl.ds(tok0 + c * 64, 64)], rows)
            pltpu.async_copy(rows, xs_hbm.at[posv.at[c]], sem).wait()

    return pl.kernel(
        body,
        mesh=mesh,
        out_type=jax.ShapeDtypeStruct((GP, D), jnp.float32),
        scratch_types=[
            pltpu.VMEM((2, 64), jnp.int32),
            pltpu.VMEM((64, D), jnp.float32),
            pltpu.SemaphoreType.DMA,
        ],
    )(pos2d, xf)


def _sc_combine(pos, wexp, ys):
    mesh = plsc.VectorSubcoreMesh(core_axis_name="c", subcore_axis_name="s")

    def body(pos_hbm, w_hbm, ys_hbm, out_hbm, i1, i2, w1s, w2s, r1, r2, o,
             sem):
        cid = lax.axis_index("c")
        sid = lax.axis_index("s")
        wid = sid * 2 + cid
        base = wid * 64
        pltpu.sync_copy(pos_hbm.at[pl.ds(base, 64)], i1)
        pltpu.sync_copy(pos_hbm.at[pl.ds(T + base, 64)], i2)
        pltpu.sync_copy(w_hbm.at[pl.ds(base, 64)], w1s)
        pltpu.sync_copy(w_hbm.at[pl.ds(T + base, 64)], w2s)
        lanes = pl.ds(0, 16)
        for c in range(2):
            pltpu.async_copy(ys_hbm.at[i1.at[pl.ds(c * 32, 32)]], r1,
                             sem).wait()
            pltpu.async_copy(ys_hbm.at[i2.at[pl.ds(c * 32, 32)]], r2,
                             sem).wait()

            def tok(t, carry):
                a1 = w1s[c * 32 + t, lanes]
                a2 = w2s[c * 32 + t, lanes]
                for v in range(D // 16):
                    sl = pl.ds(v * 16, 16)
                    o[t, sl] = a1 * r1[t, sl] + a2 * r2[t, sl]
                return carry

            lax.fori_loop(0, 32, tok, 0)
            pltpu.sync_copy(o, out_hbm.at[pl.ds(base + c * 32, 32)])

    return pl.kernel(
        body,
        mesh=mesh,
        out_type=jax.ShapeDtypeStruct((T, D), jnp.float32),
        scratch_types=[
            pltpu.VMEM((64,), jnp.int32),
            pltpu.VMEM((64,), jnp.int32),
            pltpu.VMEM((64, 16), jnp.float32),
            pltpu.VMEM((64, 16), jnp.float32),
            pltpu.VMEM((32, D), jnp.float32),
            pltpu.VMEM((32, D), jnp.float32),
            pltpu.VMEM((32, D), jnp.float32),
            pltpu.SemaphoreType.DMA,
        ],
    )(pos, wexp, ys)


def kernel(x, gate_W, gate_b, Wg, bg, Wu, bu, Wd, bd):
    B, S, _ = x.shape
    xf = x.reshape(T, D)
    pos, wexp, be = _route(xf, gate_W, gate_b)

    xs = _sc_disperse(pos.reshape(64, 64), xf)
    ys = _ffn(be, xs, Wg, bg, Wu, bu, Wd, bd)
    out = _sc_combine(pos, wexp, ys)
    return out.reshape(B, S, D)


# speedup vs baseline: 1.1741x; 1.0029x over previous
import jax
import jax.numpy as jnp
from jax import lax
from jax.experimental import pallas as pl
from jax.experimental.pallas import tpu as pltpu
from jax.experimental.pallas import tpu_sc as plsc

D = 1024
H = 2048
E = 16
T = 2048
P = 2 * T
BT = 256
G = 31
GP = G * BT


def _route_kernel(x_ref, gw_ref, gb_ref, pos_ref, w_ref, be_ref):
    f32 = jnp.float32
    xf = x_ref[...]
    logits = jax.lax.dot_general(
        xf, gw_ref[...], (((1,), (1,)), ((), ())),
        preferred_element_type=f32) + gb_ref[...][None, :]
    s = (logits > 1.0).astype(jnp.int32)
    e_iota = jax.lax.broadcasted_iota(jnp.int32, (T, E), 1)
    f1 = e_iota + (1 - s) * E
    m1 = jnp.min(f1, axis=1)
    idx1 = jnp.where(m1 < E, m1, 0)
    v1 = (m1 < E).astype(f32)
    f2 = f1 + jnp.where(e_iota == idx1[:, None], 16 * E, 0)
    m2 = jnp.min(f2, axis=1)
    idx2 = jnp.where(m2 < E, m2, m2 - E)
    v2 = (m2 < E).astype(f32)
    w1 = 1.0 / (1.0 + jnp.exp(v2 - v1))
    w2 = 1.0 - w1

    oh1 = (idx1[:, None] == e_iota).astype(f32)
    oh2 = (idx2[:, None] == e_iota).astype(f32)
    oh = jnp.concatenate([oh1, oh2], axis=0)

    RB = 256
    nb = P // RB
    ltb = (jax.lax.broadcasted_iota(jnp.int32, (RB, RB), 1)
           < jax.lax.broadcasted_iota(jnp.int32, (RB, RB), 0)).astype(f32)
    parts = []
    sums = []
    for b in range(nb):
        ohb = oh[b * RB:(b + 1) * RB]
        parts.append(jnp.dot(ltb, ohb, preferred_element_type=f32))
        sums.append(jnp.sum(ohb, axis=0)[None, :])
    excl_in = jnp.concatenate(parts, axis=0)
    bsums = jnp.concatenate(sums, axis=0)
    ltn = (jax.lax.broadcasted_iota(jnp.int32, (nb, nb), 1)
           < jax.lax.broadcasted_iota(jnp.int32, (nb, nb), 0)).astype(f32)
    bpre = jnp.dot(ltn, bsums, preferred_element_type=f32)
    bases = [jnp.broadcast_to(bpre[b][None, :], (RB, E)) for b in range(nb)]
    excl = excl_in + jnp.concatenate(bases, axis=0)
    rank = jnp.sum(oh * excl, axis=1)

    counts = jnp.sum(oh, axis=0)
    nblk = jnp.floor((counts + (BT - 1)) * (1.0 / BT))
    lte = (jax.lax.broadcasted_iota(jnp.int32, (E, E), 1)
           < jax.lax.broadcasted_iota(jnp.int32, (E, E), 0)).astype(f32)
    blk_start = jnp.dot(lte, nblk[:, None],
                        preferred_element_type=f32)[:, 0]
    pad_off = blk_start * BT
    pos = rank + jnp.sum(oh * pad_off[None, :], axis=1)

    blk_end = blk_start + nblk
    b_iota = jax.lax.broadcasted_iota(jnp.int32, (64, E), 0).astype(f32)
    be = jnp.sum((blk_end[None, :] <= b_iota).astype(f32), axis=1)
    be = jnp.minimum(be, float(E - 1))
    used = jnp.sum(nblk)
    slot = jax.lax.broadcasted_iota(jnp.int32, (64,), 0)
    be = jnp.where(slot == 63, used, be)

    pos_ref[...] = pos.astype(jnp.int32)
    wcat = jnp.concatenate([w1, w2], axis=0)
    w_ref[...] = jnp.broadcast_to(wcat[:, None], (P, 16))
    be_ref[...] = be.astype(jnp.int32)


def _route(xf, gate_W, gate_b):
    return pl.pallas_call(
        _route_kernel,
        out_shape=(
            jax.ShapeDtypeStruct((P,), jnp.int32),
            jax.ShapeDtypeStruct((P, 16), jnp.float32),
            jax.ShapeDtypeStruct((64,), jnp.int32),
        ),
    )(xf, gate_W, gate_b)


def _cb(b, be):
    return jnp.minimum(b, be[63] - 1)


def _ffn_kernel(be_ref, xs_ref, wg_ref, bg_ref, wu_ref, bu_ref,
                wd_ref, bd_ref, ys_ref):
    f32 = jnp.float32
    xb = xs_ref[...]
    h = jax.lax.dot_general(
        xb, wg_ref[0], (((1,), (1,)), ((), ())),
        preferred_element_type=f32) + bg_ref[0]
    sp = (h > 1.0).astype(f32)
    up = jax.lax.dot_general(
        xb, wu_ref[0], (((1,), (1,)), ((), ())),
        preferred_element_type=f32) + bu_ref[0]
    prod = sp * up
    ys_ref[...] = jax.lax.dot_general(
        prod, wd_ref[0], (((1,), (1,)), ((), ())),
        preferred_element_type=f32) + bd_ref[0]


def _ffn_kernel_skip(be_ref, xs_ref, wg_ref, bg_ref, wu_ref, bu_ref,
                     wd_ref, bd_ref, ys_ref):
    @pl.when(pl.program_id(0) < be_ref[63])
    def _():
        _ffn_kernel(be_ref, xs_ref, wg_ref, bg_ref, wu_ref, bu_ref,
                    wd_ref, bd_ref, ys_ref)


def _ffn(be, xs, Wg, bg, Wu, bu, Wd, bd):
    grid_spec = pltpu.PrefetchScalarGridSpec(
        num_scalar_prefetch=1,
        grid=(G,),
        in_specs=[
            pl.BlockSpec((BT, D), lambda b, be: (_cb(b, be), 0)),
            pl.BlockSpec((1, H, D), lambda b, be: (be[_cb(b, be)], 0, 0)),
            pl.BlockSpec((1, 1, H), lambda b, be: (be[_cb(b, be)], 0, 0)),
            pl.BlockSpec((1, H, D), lambda b, be: (be[_cb(b, be)], 0, 0)),
            pl.BlockSpec((1, 1, H), lambda b, be: (be[_cb(b, be)], 0, 0)),
            pl.BlockSpec((1, D, H), lambda b, be: (be[_cb(b, be)], 0, 0)),
            pl.BlockSpec((1, 1, D), lambda b, be: (be[_cb(b, be)], 0, 0)),
        ],
        out_specs=pl.BlockSpec((BT, D), lambda b, be: (_cb(b, be), 0)),
    )
    return pl.pallas_call(
        _ffn_kernel_skip,
        grid_spec=grid_spec,
        out_shape=jax.ShapeDtypeStruct((GP, D), jnp.float32),
        compiler_params=pltpu.CompilerParams(
            dimension_semantics=("arbitrary",),
            vmem_limit_bytes=64 * 1024 * 1024,
        ),
    )(be, xs, Wg, bg.reshape(E, 1, H),
      Wu, bu.reshape(E, 1, H),
      Wd, bd.reshape(E, 1, D))


def _sc_disperse(pos2d, xf):
    mesh = plsc.VectorSubcoreMesh(core_axis_name="c", subcore_axis_name="s")

    def body(pos_hbm, x_hbm, xs_hbm, posv, rows, sem):
        cid = lax.axis_index("c")
        sid = lax.axis_index("s")
        wid = sid * 2 + cid
        tok0 = (wid * 128) % T
        pltpu.sync_copy(pos_hbm.at[pl.ds(wid * 2, 2)], posv)
        for c in range(2):
            pltpu.sync_copy(x_hbm.at[pl.ds(tok0 + c * 64, 64)], rows)
            pltpu.async_copy(rows, xs_hbm.at[posv.at[c]], sem).wait()

    return pl.kernel(
        body,
        mesh=mesh,
        out_type=jax.ShapeDtypeStruct((GP, D), jnp.float32),
        scratch_types=[
            pltpu.VMEM((2, 64), jnp.int32),
            pltpu.VMEM((64, D), jnp.float32),
            pltpu.SemaphoreType.DMA,
        ],
    )(pos2d, xf)


def _sc_combine(pos, wexp, ys):
    mesh = plsc.VectorSubcoreMesh(core_axis_name="c", subcore_axis_name="s")

    def body(pos_hbm, w_hbm, ys_hbm, out_hbm, i1, i2, w1s, w2s, r1, r2, o,
             sem):
        cid = lax.axis_index("c")
        sid = lax.axis_index("s")
        wid = sid * 2 + cid
        base = wid * 64
        pltpu.sync_copy(pos_hbm.at[pl.ds(base, 64)], i1)
        pltpu.sync_copy(pos_hbm.at[pl.ds(T + base, 64)], i2)
        pltpu.sync_copy(w_hbm.at[pl.ds(base, 64)], w1s)
        pltpu.sync_copy(w_hbm.at[pl.ds(T + base, 64)], w2s)
        lanes = pl.ds(0, 16)
        for c in range(2):
            pltpu.async_copy(ys_hbm.at[i1.at[pl.ds(c * 32, 32)]], r1,
                             sem).wait()
            pltpu.async_copy(ys_hbm.at[i2.at[pl.ds(c * 32, 32)]], r2,
                             sem).wait()

            def tok(t, carry):
                a1 = w1s[c * 32 + t, lanes]
                a2 = w2s[c * 32 + t, lanes]
                for v in range(D // 16):
                    sl = pl.ds(v * 16, 16)
                    o[t, sl] = a1 * r1[t, sl] + a2 * r2[t, sl]
                return carry

            lax.fori_loop(0, 32, tok, 0)
            pltpu.sync_copy(o, out_hbm.at[pl.ds(base + c * 32, 32)])

    return pl.kernel(
        body,
        mesh=mesh,
        out_type=jax.ShapeDtypeStruct((T, D), jnp.float32),
        scratch_types=[
            pltpu.VMEM((64,), jnp.int32),
            pltpu.VMEM((64,), jnp.int32),
            pltpu.VMEM((64, 16), jnp.float32),
            pltpu.VMEM((64, 16), jnp.float32),
            pltpu.VMEM((32, D), jnp.float32),
            pltpu.VMEM((32, D), jnp.float32),
            pltpu.VMEM((32, D), jnp.float32),
            pltpu.SemaphoreType.DMA,
        ],
    )(pos, wexp, ys)


def kernel(x, gate_W, gate_b, Wg, bg, Wu, bu, Wd, bd):
    B, S, _ = x.shape
    xf = x.reshape(T, D)
    pos, wexp, be = _route(xf, gate_W, gate_b)

    xs = _sc_disperse(pos.reshape(64, 64), xf)
    ys = _ffn(be, xs, Wg, bg, Wu, bu, Wd, bd)
    out = _sc_combine(pos, wexp, ys)
    return out.reshape(B, S, D)
